# baseline (speedup 1.0000x reference)
import functools

import jax
import jax.numpy as jnp
from jax import lax
from jax.experimental import pallas as pl
from jax.experimental.pallas import tpu as pltpu
from jax.experimental.pallas import tpu_sc as plsc

VOCAB = 100000
EMBED = 128
BATCH = 4096
CTX = 20

LANES = 16
NW = 32
BPW = BATCH // NW
CHUNK_B = 4
NCHUNK = BPW // CHUNK_B
IPC = CHUNK_B * CTX

VT = 8192
BT = 256


def _pool_sc(ctx_idx, table):
    mesh = plsc.VectorSubcoreMesh(core_axis_name="c", subcore_axis_name="s")

    @functools.partial(
        pl.kernel,
        out_type=jax.ShapeDtypeStruct((BATCH, EMBED), jnp.float32),
        mesh=mesh,
        scratch_types=[
            pltpu.VMEM((NCHUNK, IPC), jnp.int32),
            pltpu.VMEM((IPC, EMBED), jnp.float32),
            pltpu.VMEM((BPW, EMBED), jnp.float32),
            pltpu.SemaphoreType.DMA,
        ],
    )
    def pool(idx_hbm, table_hbm, out_hbm, idx_v, rows_v, pooled_v, sem):
        wid = lax.axis_index("s") * mesh.num_cores + lax.axis_index("c")
        pltpu.sync_copy(idx_hbm.at[wid], idx_v)

        def body(c, carry):
            pltpu.async_copy(table_hbm.at[idx_v.at[c]], rows_v, sem).wait()
            for b in range(CHUNK_B):
                for r in range(EMBED // LANES):
                    sl = pl.ds(r * LANES, LANES)
                    acc = rows_v[b * CTX, sl]
                    for j in range(1, CTX):
                        acc = acc + rows_v[b * CTX + j, sl]
                    pooled_v[c * CHUNK_B + b, sl] = acc * (1.0 / CTX)
            return carry

        lax.fori_loop(0, NCHUNK, body, 0)
        pltpu.sync_copy(pooled_v, out_hbm.at[pl.ds(wid * BPW, BPW)])

    return pool(ctx_idx, table)


def _project_tc(pooled, W_out, b_out):
    def mm(p_ref, w_ref, b_ref, o_ref):
        o_ref[...] = lax.dot_general(
            p_ref[...], w_ref[...], (((1,), (1,)), ((), ())),
            preferred_element_type=jnp.float32,
        ) + b_ref[...]

    return pl.pallas_call(
        mm,
        grid=(pl.cdiv(VOCAB, VT), BATCH // BT),
        in_specs=[
            pl.BlockSpec((BT, EMBED), lambda j, i: (i, 0)),
            pl.BlockSpec((VT, EMBED), lambda j, i: (j, 0)),
            pl.BlockSpec((1, VT), lambda j, i: (0, j)),
        ],
        out_specs=pl.BlockSpec((BT, VT), lambda j, i: (i, j)),
        out_shape=jax.ShapeDtypeStruct((BATCH, VOCAB), jnp.float32),
    )(pooled, W_out, b_out.reshape(1, VOCAB))


def kernel(context, embeddings, W_out, b_out):
    idx = context.astype(jnp.int32).reshape(NW, NCHUNK, IPC)
    pooled = _pool_sc(idx, embeddings)
    return _project_tc(pooled, W_out, b_out)


# speedup vs baseline: 1.0101x; 1.0101x over previous
import functools

import jax
import jax.numpy as jnp
from jax import lax
from jax.experimental import pallas as pl
from jax.experimental.pallas import tpu as pltpu
from jax.experimental.pallas import tpu_sc as plsc

VOCAB = 100000
EMBED = 128
BATCH = 4096
CTX = 20

LANES = 16
NW = 32
BPW = BATCH // NW
CHUNK_B = 4
NCHUNK = BPW // CHUNK_B
IPC = CHUNK_B * CTX

VT = 8192
BT = 256


def _pool_sc(ctx_idx, table):
    mesh = plsc.VectorSubcoreMesh(core_axis_name="c", subcore_axis_name="s")

    @functools.partial(
        pl.kernel,
        out_type=jax.ShapeDtypeStruct((BATCH, EMBED), jnp.float32),
        mesh=mesh,
        scratch_types=[
            pltpu.VMEM((NCHUNK, IPC), jnp.int32),
            pltpu.VMEM((IPC, EMBED), jnp.float32),
            pltpu.VMEM((BPW, EMBED), jnp.float32),
            pltpu.SemaphoreType.DMA,
        ],
    )
    def pool(idx_hbm, table_hbm, out_hbm, idx_v, rows_v, pooled_v, sem):
        wid = lax.axis_index("s") * mesh.num_cores + lax.axis_index("c")
        pltpu.sync_copy(idx_hbm.at[wid], idx_v)

        def body(c, carry):
            pltpu.async_copy(table_hbm.at[idx_v.at[c]], rows_v, sem).wait()
            for b in range(CHUNK_B):
                for r in range(EMBED // LANES):
                    sl = pl.ds(r * LANES, LANES)
                    acc = rows_v[b * CTX, sl]
                    for j in range(1, CTX):
                        acc = acc + rows_v[b * CTX + j, sl]
                    pooled_v[c * CHUNK_B + b, sl] = acc * (1.0 / CTX)
            return carry

        lax.fori_loop(0, NCHUNK, body, 0)
        pltpu.sync_copy(pooled_v, out_hbm.at[pl.ds(wid * BPW, BPW)])

    return pool(ctx_idx, table)


NB = BATCH // BT
NV = (VOCAB + VT - 1) // VT
TAIL_START = (NV - 1) * VT
TAIL = VOCAB - TAIL_START
NSTEPS = NV * NB
NBUF = 4
NTB = 2


def _project_tc(pooled, W_out, b_out):
    def mm(p_ref, w_ref, b_ref, bt_ref, o_ref, acc, tacc, sems, tsems):
        j = pl.program_id(0)
        i = pl.program_id(1)
        step = j * NB + i
        row = i * BT

        @pl.when(j < NV - 1)
        def _():
            for k in range(NBUF):

                @pl.when(lax.rem(step, NBUF) == k)
                def _():
                    @pl.when(step >= NBUF)
                    def _():
                        pltpu.make_async_copy(
                            acc.at[k],
                            o_ref.at[pl.ds(0, BT), pl.ds(0, VT)],
                            sems.at[k]).wait()

                    acc[k] = lax.dot_general(
                        p_ref[...], w_ref[...], (((1,), (1,)), ((), ())),
                        preferred_element_type=jnp.float32,
                    ) + b_ref[...]

                    pltpu.make_async_copy(
                        acc.at[k],
                        o_ref.at[pl.ds(row, BT),
                                 pl.ds(pl.multiple_of(j * VT, VT), VT)],
                        sems.at[k]).start()

        @pl.when(j == NV - 1)
        def _():
            for t in range(NTB):

                @pl.when(lax.rem(i, NTB) == t)
                def _():
                    @pl.when(i >= NTB)
                    def _():
                        pltpu.make_async_copy(
                            tacc.at[t],
                            o_ref.at[pl.ds(0, BT),
                                     pl.ds(TAIL_START, TAIL)],
                            tsems.at[t]).wait()

                    tacc[t] = lax.dot_general(
                        p_ref[...], w_ref[pl.ds(0, TAIL), :],
                        (((1,), (1,)), ((), ())),
                        preferred_element_type=jnp.float32,
                    ) + bt_ref[...]

                    pltpu.make_async_copy(
                        tacc.at[t],
                        o_ref.at[pl.ds(row, BT), pl.ds(TAIL_START, TAIL)],
                        tsems.at[t]).start()

        @pl.when(step == NSTEPS - 1)
        def _():
            for m in range(NBUF):
                pltpu.make_async_copy(
                    acc.at[m], o_ref.at[pl.ds(0, BT), pl.ds(0, VT)],
                    sems.at[m]).wait()
            for t in range(NTB):
                pltpu.make_async_copy(
                    tacc.at[t],
                    o_ref.at[pl.ds(0, BT), pl.ds(TAIL_START, TAIL)],
                    tsems.at[t]).wait()

    return pl.pallas_call(
        mm,
        grid=(NV, NB),
        in_specs=[
            pl.BlockSpec((BT, EMBED), lambda j, i: (i, 0)),
            pl.BlockSpec((VT, EMBED), lambda j, i: (j, 0)),
            pl.BlockSpec((1, VT), lambda j, i: (0, j)),
            pl.BlockSpec((1, TAIL), lambda j, i: (0, 0)),
        ],
        out_specs=pl.BlockSpec(memory_space=pl.ANY),
        out_shape=jax.ShapeDtypeStruct((BATCH, VOCAB), jnp.float32),
        scratch_shapes=[
            pltpu.VMEM((NBUF, BT, VT), jnp.float32),
            pltpu.VMEM((NTB, BT, TAIL), jnp.float32),
            pltpu.SemaphoreType.DMA((NBUF,)),
            pltpu.SemaphoreType.DMA((NTB,)),
        ],
    )(pooled, W_out, b_out.reshape(1, VOCAB),
      lax.slice(b_out, (TAIL_START,), (VOCAB,)).reshape(1, TAIL))


def kernel(context, embeddings, W_out, b_out):
    idx = context.astype(jnp.int32).reshape(NW, NCHUNK, IPC)
    pooled = _pool_sc(idx, embeddings)
    return _project_tc(pooled, W_out, b_out)


# speedup vs baseline: 1.0133x; 1.0031x over previous
import functools

import jax
import jax.numpy as jnp
from jax import lax
from jax.experimental import pallas as pl
from jax.experimental.pallas import tpu as pltpu
from jax.experimental.pallas import tpu_sc as plsc

VOCAB = 100000
EMBED = 128
BATCH = 4096
CTX = 20

LANES = 16
NW = 32
BPW = BATCH // NW
CHUNK_B = 4
NCHUNK = BPW // CHUNK_B
IPC = CHUNK_B * CTX

VT = 512


def _pool_sc(ctx_idx, table):
    mesh = plsc.VectorSubcoreMesh(core_axis_name="c", subcore_axis_name="s")

    @functools.partial(
        pl.kernel,
        out_type=jax.ShapeDtypeStruct((BATCH, EMBED), jnp.float32),
        mesh=mesh,
        scratch_types=[
            pltpu.VMEM((NCHUNK, IPC), jnp.int32),
            pltpu.VMEM((2, IPC, EMBED), jnp.float32),
            pltpu.VMEM((BPW, EMBED), jnp.float32),
            pltpu.SemaphoreType.DMA((2,)),
        ],
    )
    def pool(idx_hbm, table_hbm, out_hbm, idx_v, rows_v, pooled_v, sems):
        wid = lax.axis_index("s") * mesh.num_cores + lax.axis_index("c")
        pltpu.sync_copy(idx_hbm.at[wid], idx_v)

        def accumulate(slot, c):
            for b in range(CHUNK_B):
                for r in range(EMBED // LANES):
                    sl = pl.ds(r * LANES, LANES)
                    acc = rows_v[slot, b * CTX, sl]
                    for j in range(1, CTX):
                        acc = acc + rows_v[slot, b * CTX + j, sl]
                    pooled_v[c * CHUNK_B + b, sl] = acc * (1.0 / CTX)

        pltpu.make_async_copy(
            table_hbm.at[idx_v.at[0]], rows_v.at[0], sems.at[0]).start()

        def body(c2, carry):
            c0 = c2 * 2
            pltpu.make_async_copy(
                table_hbm.at[idx_v.at[c0]], rows_v.at[0], sems.at[0]).wait()
            pltpu.make_async_copy(
                table_hbm.at[idx_v.at[c0 + 1]], rows_v.at[1],
                sems.at[1]).start()
            accumulate(0, c0)
            pltpu.make_async_copy(
                table_hbm.at[idx_v.at[c0 + 1]], rows_v.at[1],
                sems.at[1]).wait()

            @pl.when(c2 + 1 < NCHUNK // 2)
            def _():
                pltpu.make_async_copy(
                    table_hbm.at[idx_v.at[c0 + 2]], rows_v.at[0],
                    sems.at[0]).start()

            accumulate(1, c0 + 1)
            return carry

        lax.fori_loop(0, NCHUNK // 2, body, 0)
        pltpu.sync_copy(pooled_v, out_hbm.at[pl.ds(wid * BPW, BPW)])

    return pool(ctx_idx, table)


def _project_tc(pooled, W_out, b_out):
    def mm(p_ref, w_ref, b_ref, o_ref):
        o_ref[...] = lax.dot_general(
            p_ref[...], w_ref[...], (((1,), (1,)), ((), ())),
            preferred_element_type=jnp.float32,
        ) + b_ref[...]

    return pl.pallas_call(
        mm,
        grid=(pl.cdiv(VOCAB, VT),),
        in_specs=[
            pl.BlockSpec((BATCH, EMBED), lambda j: (0, 0)),
            pl.BlockSpec((VT, EMBED), lambda j: (j, 0)),
            pl.BlockSpec((1, VT), lambda j: (0, j)),
        ],
        out_specs=pl.BlockSpec((BATCH, VT), lambda j: (0, j)),
        out_shape=jax.ShapeDtypeStruct((BATCH, VOCAB), jnp.float32),
    )(pooled, W_out, b_out.reshape(1, VOCAB))


def kernel(context, embeddings, W_out, b_out):
    idx = context.astype(jnp.int32).reshape(NW, NCHUNK, IPC)
    pooled = _pool_sc(idx, embeddings)
    return _project_tc(pooled, W_out, b_out)


# speedup vs baseline: 1.0270x; 1.0136x over previous
import functools

import jax
import jax.numpy as jnp
from jax import lax
from jax.experimental import pallas as pl
from jax.experimental.pallas import tpu as pltpu
from jax.experimental.pallas import tpu_sc as plsc

VOCAB = 100000
EMBED = 128
BATCH = 4096
CTX = 20

LANES = 16
NW = 32
BPW = BATCH // NW
CHUNK_B = 4
NCHUNK = BPW // CHUNK_B
IPC = CHUNK_B * CTX

VT = 1024


def _pool_sc(ctx_idx, table):
    mesh = plsc.VectorSubcoreMesh(core_axis_name="c", subcore_axis_name="s")

    @functools.partial(
        pl.kernel,
        out_type=jax.ShapeDtypeStruct((BATCH, EMBED), jnp.float32),
        mesh=mesh,
        scratch_types=[
            pltpu.VMEM((NCHUNK, IPC), jnp.int32),
            pltpu.VMEM((2, IPC, EMBED), jnp.float32),
            pltpu.VMEM((BPW, EMBED), jnp.float32),
            pltpu.SemaphoreType.DMA((2,)),
        ],
    )
    def pool(idx_hbm, table_hbm, out_hbm, idx_v, rows_v, pooled_v, sems):
        wid = lax.axis_index("s") * mesh.num_cores + lax.axis_index("c")
        pltpu.sync_copy(idx_hbm.at[wid], idx_v)

        def accumulate(slot, c):
            for b in range(CHUNK_B):
                for r in range(EMBED // LANES):
                    sl = pl.ds(r * LANES, LANES)
                    acc = rows_v[slot, b * CTX, sl]
                    for j in range(1, CTX):
                        acc = acc + rows_v[slot, b * CTX + j, sl]
                    pooled_v[c * CHUNK_B + b, sl] = acc * (1.0 / CTX)

        pltpu.make_async_copy(
            table_hbm.at[idx_v.at[0]], rows_v.at[0], sems.at[0]).start()

        def body(c2, carry):
            c0 = c2 * 2
            pltpu.make_async_copy(
                table_hbm.at[idx_v.at[c0]], rows_v.at[0], sems.at[0]).wait()
            pltpu.make_async_copy(
                table_hbm.at[idx_v.at[c0 + 1]], rows_v.at[1],
                sems.at[1]).start()
            accumulate(0, c0)
            pltpu.make_async_copy(
                table_hbm.at[idx_v.at[c0 + 1]], rows_v.at[1],
                sems.at[1]).wait()

            @pl.when(c2 + 1 < NCHUNK // 2)
            def _():
                pltpu.make_async_copy(
                    table_hbm.at[idx_v.at[c0 + 2]], rows_v.at[0],
                    sems.at[0]).start()

            accumulate(1, c0 + 1)
            return carry

        lax.fori_loop(0, NCHUNK // 2, body, 0)
        pltpu.sync_copy(pooled_v, out_hbm.at[pl.ds(wid * BPW, BPW)])

    return pool(ctx_idx, table)


def _project_tc(pooled, W_out, b_out):
    def mm(p_ref, w_ref, b_ref, o_ref):
        o_ref[...] = lax.dot_general(
            p_ref[...], w_ref[...], (((1,), (1,)), ((), ())),
            preferred_element_type=jnp.float32,
        ) + b_ref[...]

    return pl.pallas_call(
        mm,
        grid=(pl.cdiv(VOCAB, VT),),
        in_specs=[
            pl.BlockSpec((BATCH, EMBED), lambda j: (0, 0)),
            pl.BlockSpec((VT, EMBED), lambda j: (j, 0)),
            pl.BlockSpec((1, VT), lambda j: (0, j)),
        ],
        out_specs=pl.BlockSpec((BATCH, VT), lambda j: (0, j)),
        out_shape=jax.ShapeDtypeStruct((BATCH, VOCAB), jnp.float32),
    )(pooled, W_out, b_out.reshape(1, VOCAB))


def kernel(context, embeddings, W_out, b_out):
    idx = context.astype(jnp.int32).reshape(NW, NCHUNK, IPC)
    pooled = _pool_sc(idx, embeddings)
    return _project_tc(pooled, W_out, b_out)
